# baseline (speedup 1.0000x reference)
import jax
import jax.numpy as jnp
from jax import lax
from jax.experimental import pallas as pl
from jax.experimental.pallas import tpu as pltpu
from jax.experimental.pallas import tpu_sc as plsc

N = 10000
D_FEAT = 128
H = 20
HP = 32
Z = 4
ZP = 16
DW = 16
E = 320000
NC, NS = 2, 16
CH = 125
KF = 8
NCHUNK = E // CH
C0 = 88
C1 = 72
T_ROWS = N // NS

_ROW_BLK = N
_GRID = N // _ROW_BLK


def _mesh():
    return plsc.VectorSubcoreMesh(
        core_axis_name="c", subcore_axis_name="s", num_cores=NC, num_subcores=NS
    )


_SC_PARAMS = pltpu.CompilerParams(use_tc_tiling_on_sc=False)


def _stage_chunks(cid, sid, e3, row, idx_ref):
    @pl.when(cid == 0)
    def _():
        pltpu.sync_copy(e3.at[row, pl.ds(sid * C0, C0)], idx_ref.at[pl.ds(0, C0)])

    @pl.when(cid == 1)
    def _():
        pltpu.sync_copy(
            e3.at[row, pl.ds(NS * C0 + sid * C1, C1)], idx_ref.at[pl.ds(0, C1)]
        )


_DCH = 128
_DNCH = E // _DCH


def _make_sc_degree():
    def body(e2, ones_c, zrow, out, idx_d, ones_v, stage, acc, sem):
        cid = lax.axis_index("c")
        sid = lax.axis_index("s")
        t = cid * NS + sid
        cnt = 78 + (t < 4).astype(jnp.int32)
        base = (78 * t + jnp.minimum(t, 4)) * _DCH
        pltpu.sync_copy(zrow, stage)
        pltpu.sync_copy(stage, acc.at[pl.ds(sid * T_ROWS, T_ROWS)])
        pltpu.sync_copy(ones_c, ones_v)

        @pl.when(t < 4)
        def _():
            pltpu.sync_copy(
                e2.at[1, pl.ds(base, 79 * _DCH)], idx_d.at[pl.ds(0, 79 * _DCH)]
            )

        @pl.when(t >= 4)
        def _():
            pltpu.sync_copy(
                e2.at[1, pl.ds(base, 78 * _DCH)], idx_d.at[pl.ds(0, 78 * _DCH)]
            )

        plsc.subcore_barrier()

        @pl.loop(0, 10)
        def _grp(g):
            for b in range(KF):
                c = g * KF + b

                @pl.when(c < cnt)
                def _():
                    pltpu.async_copy(
                        ones_v, acc.at[idx_d.at[pl.ds(c * _DCH, _DCH)]], sem, add=True
                    )

            for b in range(KF):
                c = g * KF + b

                @pl.when(c < cnt)
                def _():
                    pltpu.make_async_copy(
                        ones_v, acc.at[idx_d.at[pl.ds(c * _DCH, _DCH)]], sem
                    ).wait()

        plsc.subcore_barrier()
        pltpu.sync_copy(acc.at[pl.ds(sid * T_ROWS, T_ROWS)], stage)
        pltpu.sync_copy(stage, out.at[cid, pl.ds(sid * T_ROWS, T_ROWS)])

    return pl.kernel(
        body,
        out_type=jax.ShapeDtypeStruct((NC, N, DW), jnp.float32),
        mesh=_mesh(),
        compiler_params=_SC_PARAMS,
        scratch_types=[
            pltpu.VMEM((79 * _DCH,), jnp.int32),
            pltpu.VMEM((_DCH, DW), jnp.float32),
            pltpu.VMEM((T_ROWS, DW), jnp.float32),
            pltpu.VMEM_SHARED((N, DW), jnp.float32),
            pltpu.SemaphoreType.DMA,
        ],
    )


_MAGIC = 0x5F3759DF


def _rsqrt16(deg):
    i = lax.bitcast_convert_type(deg, jnp.int32)
    magic = jnp.full((16,), _MAGIC, jnp.int32)
    y = lax.bitcast_convert_type(
        magic - lax.shift_right_arithmetic(i, jnp.full((16,), 1, jnp.int32)),
        jnp.float32,
    )
    half = jnp.full((16,), 0.5, jnp.float32)
    threehalves = jnp.full((16,), 1.5, jnp.float32)
    for _ in range(3):
        y = y * (threehalves - half * deg * y * y)
    return y


def _make_sc_scatter(d):
    def body(e3, ulin, deg2, zrow, s_out, vtab, dinv_out,
             idx_s, idx_d, rows, ustage, vstage, dga, dgb, dstage, acc,
             semA, semB, semS):
        cid = lax.axis_index("c")
        sid = lax.axis_index("s")
        ngrp = jnp.where(cid == 0, C0 // KF, C1 // KF)
        row0 = sid * T_ROWS
        _stage_chunks(cid, sid, e3, 0, idx_s)
        _stage_chunks(cid, sid, e3, 1, idx_d)

        @pl.loop(0, 5)
        def _sub(s):
            r0 = row0 + s * 125
            a0 = sid * T_ROWS + s * 125
            pltpu.sync_copy(ulin.at[pl.ds(r0, 125)], ustage)
            pltpu.sync_copy(deg2.at[0, pl.ds(r0, 125)], dga)
            pltpu.sync_copy(deg2.at[1, pl.ds(r0, 125)], dgb)

            @pl.loop(0, 125)
            def _row(r):
                deg = 1.0 + dga[r] + dgb[r]
                y = _rsqrt16(deg)
                dstage[r] = y
                for j in range(d // 16):
                    vstage[r, pl.ds(16 * j, 16)] = ustage[r, pl.ds(16 * j, 16)] * y

            pltpu.sync_copy(vstage, vtab.at[cid, pl.ds(r0, 125)])

            @pl.when(cid == 0)
            def _():
                pltpu.sync_copy(vstage, acc.at[pl.ds(a0, 125)])
                pltpu.sync_copy(dstage, dinv_out.at[pl.ds(r0, 125)])

            @pl.when(cid == 1)
            def _():
                pltpu.sync_copy(zrow, acc.at[pl.ds(a0, 125)])

        plsc.subcore_barrier()
        table = vtab.at[cid]

        def fire_gathers(g, buf, sem):
            for b in range(KF):
                pltpu.async_copy(table.at[idx_s.at[g * KF + b]], buf.at[b], sem)

        def drain_gathers(g, buf, sem):
            for b in range(KF):
                pltpu.make_async_copy(
                    table.at[idx_s.at[g * KF + b]], buf.at[b], sem
                ).wait()

        def run_scatters(g, buf):
            cps = [
                pltpu.async_copy(
                    buf.at[b], acc.at[idx_d.at[g * KF + b]], semS, add=True
                )
                for b in range(KF)
            ]
            for cp in cps:
                cp.wait()

        def step(g, buf, sem, obuf, osem):
            @pl.when(g + 1 < ngrp)
            def _():
                fire_gathers(g + 1, obuf, osem)

            drain_gathers(g, buf, sem)
            run_scatters(g, buf)

        fire_gathers(0, rows.at[0], semA)

        @pl.loop(0, ngrp)
        def _grp(g):
            @pl.when(g % 2 == 0)
            def _():
                step(g, rows.at[0], semA, rows.at[1], semB)

            @pl.when(g % 2 == 1)
            def _():
                step(g, rows.at[1], semB, rows.at[0], semA)

        plsc.subcore_barrier()

        @pl.loop(0, 5)
        def _wb(s):
            pltpu.sync_copy(acc.at[pl.ds(sid * T_ROWS + s * 125, 125)], vstage)
            pltpu.sync_copy(vstage, s_out.at[cid, pl.ds(row0 + s * 125, 125)])

    return pl.kernel(
        body,
        out_type=(
            jax.ShapeDtypeStruct((NC, N, d), jnp.float32),
            jax.ShapeDtypeStruct((NC, N, d), jnp.float32),
            jax.ShapeDtypeStruct((N, DW), jnp.float32),
        ),
        mesh=_mesh(),
        compiler_params=_SC_PARAMS,
        scratch_types=[
            pltpu.VMEM((C0, CH), jnp.int32),
            pltpu.VMEM((C0, CH), jnp.int32),
            pltpu.VMEM((2, KF, CH, d), jnp.float32),
            pltpu.VMEM((125, d), jnp.float32),
            pltpu.VMEM((125, d), jnp.float32),
            pltpu.VMEM((125, DW), jnp.float32),
            pltpu.VMEM((125, DW), jnp.float32),
            pltpu.VMEM((125, DW), jnp.float32),
            pltpu.VMEM_SHARED((N, d), jnp.float32),
            pltpu.SemaphoreType.DMA,
            pltpu.SemaphoreType.DMA,
            pltpu.SemaphoreType.DMA,
        ],
    )


_sc_degree = _make_sc_degree()
_sc_scatter_h = _make_sc_scatter(HP)
_sc_scatter_z = _make_sc_scatter(ZP)


def _tc_matmul(x, W1p):
    def body(x_ref, w_ref, o_ref):
        o_ref[...] = jnp.dot(x_ref[...], w_ref[...], preferred_element_type=jnp.float32)

    return pl.pallas_call(
        body,
        grid=(_GRID,),
        in_specs=[
            pl.BlockSpec((_ROW_BLK, D_FEAT), lambda i: (i, 0)),
            pl.BlockSpec((D_FEAT, HP), lambda i: (0, 0)),
        ],
        out_specs=pl.BlockSpec((_ROW_BLK, HP), lambda i: (i, 0)),
        out_shape=jax.ShapeDtypeStruct((N, HP), jnp.float32),
    )(x, W1p)


def _tc_mid(dinv16, s1, W2p, b1r):
    def body(dref, s_ref, w_ref, bias_ref, o_ref):
        dinv = dref[...][:, :1]
        ss = s_ref[...]
        h = dinv * (ss[0] + ss[1]) + bias_ref[...]
        o_ref[...] = jnp.dot(h, w_ref[...], preferred_element_type=jnp.float32)

    return pl.pallas_call(
        body,
        grid=(_GRID,),
        in_specs=[
            pl.BlockSpec((_ROW_BLK, DW), lambda i: (i, 0)),
            pl.BlockSpec((NC, _ROW_BLK, HP), lambda i: (0, i, 0)),
            pl.BlockSpec((HP, ZP), lambda i: (0, 0)),
            pl.BlockSpec((1, HP), lambda i: (0, 0)),
        ],
        out_specs=pl.BlockSpec((_ROW_BLK, ZP), lambda i: (i, 0)),
        out_shape=jax.ShapeDtypeStruct((N, ZP), jnp.float32),
    )(dinv16, s1, W2p, b1r)


def _tc_final(dinv16, s2, b2r):
    def body(dref, s_ref, bias_ref, o_ref):
        dinv = dref[...][:, :1]
        ss = s_ref[...]
        zfull = dinv * (ss[0] + ss[1]) + bias_ref[...]
        o_ref[...] = zfull[:, :Z]

    return pl.pallas_call(
        body,
        grid=(_GRID,),
        in_specs=[
            pl.BlockSpec((_ROW_BLK, DW), lambda i: (i, 0)),
            pl.BlockSpec((NC, _ROW_BLK, ZP), lambda i: (0, i, 0)),
            pl.BlockSpec((1, ZP), lambda i: (0, 0)),
        ],
        out_specs=pl.BlockSpec((_ROW_BLK, Z), lambda i: (i, 0)),
        out_shape=jax.ShapeDtypeStruct((N, Z), jnp.float32),
    )(dinv16, s2, b2r)


def kernel(x, edge_index, W1, b1, W2, b2):
    e2 = edge_index.astype(jnp.int32)
    e3 = e2.reshape(2, NCHUNK, CH)
    W1p = jnp.pad(W1, ((0, 0), (0, HP - H)))
    W2p = jnp.pad(W2, ((0, HP - H), (0, ZP - Z)))
    b1r = jnp.pad(b1, (0, HP - H)).reshape(1, HP)
    b2r = jnp.pad(b2, (0, ZP - Z)).reshape(1, ZP)
    ones_c = jnp.ones((_DCH, DW), jnp.float32)
    zdw = jnp.zeros((T_ROWS, DW), jnp.float32)
    zh = jnp.zeros((125, HP), jnp.float32)
    zz = jnp.zeros((125, ZP), jnp.float32)

    deg2 = _sc_degree(e2, ones_c, zdw)
    u1 = _tc_matmul(x, W1p)
    s1, _, dinv16 = _sc_scatter_h(e3, u1, deg2, zh)
    u2 = _tc_mid(dinv16, s1, W2p, b1r)
    s2, _, _ = _sc_scatter_z(e3, u2, deg2, zz)
    return _tc_final(dinv16, s2, b2r)


# speedup vs baseline: 1.1412x; 1.1412x over previous
import jax
import jax.numpy as jnp
from jax import lax
from jax.experimental import pallas as pl
from jax.experimental.pallas import tpu as pltpu
from jax.experimental.pallas import tpu_sc as plsc

N = 10000
D_FEAT = 128
H = 20
HP = 32
Z = 4
ZP = 16
DW = 16
E = 320000
NC, NS = 2, 16
CH = 125
KF = 8
NCHUNK = E // CH
C0 = 80
C1 = 80
T_ROWS = N // NS

_ROW_BLK = N
_GRID = N // _ROW_BLK


def _mesh():
    return plsc.VectorSubcoreMesh(
        core_axis_name="c", subcore_axis_name="s", num_cores=NC, num_subcores=NS
    )


_SC_PARAMS = pltpu.CompilerParams(use_tc_tiling_on_sc=False)


def _stage_chunks(cid, sid, e3, row, idx_ref):
    @pl.when(cid == 0)
    def _():
        pltpu.sync_copy(e3.at[row, pl.ds(sid * C0, C0)], idx_ref.at[pl.ds(0, C0)])

    @pl.when(cid == 1)
    def _():
        pltpu.sync_copy(
            e3.at[row, pl.ds(NS * C0 + sid * C1, C1)], idx_ref.at[pl.ds(0, C1)]
        )


_DCH = 128
_DNCH = E // _DCH


def _make_sc_degree():
    def body(e2, ones_c, zrow, out, idx_d, ones_v, stage, acc, sem):
        cid = lax.axis_index("c")
        sid = lax.axis_index("s")
        t = cid * NS + sid
        cnt = 78 + (t < 4).astype(jnp.int32)
        base = (78 * t + jnp.minimum(t, 4)) * _DCH
        pltpu.sync_copy(zrow, stage)
        pltpu.sync_copy(stage, acc.at[pl.ds(sid * T_ROWS, T_ROWS)])
        pltpu.sync_copy(ones_c, ones_v)

        @pl.when(t < 4)
        def _():
            pltpu.sync_copy(
                e2.at[1, pl.ds(base, 79 * _DCH)], idx_d.at[pl.ds(0, 79 * _DCH)]
            )

        @pl.when(t >= 4)
        def _():
            pltpu.sync_copy(
                e2.at[1, pl.ds(base, 78 * _DCH)], idx_d.at[pl.ds(0, 78 * _DCH)]
            )

        plsc.subcore_barrier()

        @pl.loop(0, 10)
        def _grp(g):
            for b in range(KF):
                c = g * KF + b

                @pl.when(c < cnt)
                def _():
                    pltpu.async_copy(
                        ones_v, acc.at[idx_d.at[pl.ds(c * _DCH, _DCH)]], sem, add=True
                    )

            for b in range(KF):
                c = g * KF + b

                @pl.when(c < cnt)
                def _():
                    pltpu.make_async_copy(
                        ones_v, acc.at[idx_d.at[pl.ds(c * _DCH, _DCH)]], sem
                    ).wait()

        plsc.subcore_barrier()
        pltpu.sync_copy(acc.at[pl.ds(sid * T_ROWS, T_ROWS)], stage)
        pltpu.sync_copy(stage, out.at[cid, pl.ds(sid * T_ROWS, T_ROWS)])

    return pl.kernel(
        body,
        out_type=jax.ShapeDtypeStruct((NC, N, DW), jnp.float32),
        mesh=_mesh(),
        compiler_params=_SC_PARAMS,
        scratch_types=[
            pltpu.VMEM((79 * _DCH,), jnp.int32),
            pltpu.VMEM((_DCH, DW), jnp.float32),
            pltpu.VMEM((T_ROWS, DW), jnp.float32),
            pltpu.VMEM_SHARED((N, DW), jnp.float32),
            pltpu.SemaphoreType.DMA,
        ],
    )


def _make_sc_scatter(d):
    def body(e3, table, zrow, out, idx_s, idx_d, rows, stage, acc, semA, semB, semS):
        cid = lax.axis_index("c")
        sid = lax.axis_index("s")
        ngrp = jnp.where(cid == 0, C0 // KF, C1 // KF)
        pltpu.sync_copy(zrow, stage)
        pltpu.sync_copy(stage, acc.at[pl.ds(sid * T_ROWS, T_ROWS)])
        _stage_chunks(cid, sid, e3, 0, idx_s)
        _stage_chunks(cid, sid, e3, 1, idx_d)
        plsc.subcore_barrier()

        def fire_gathers(g, buf, sem):
            for b in range(KF):
                pltpu.async_copy(table.at[idx_s.at[g * KF + b]], buf.at[b], sem)

        def drain_gathers(g, buf, sem):
            for b in range(KF):
                pltpu.make_async_copy(
                    table.at[idx_s.at[g * KF + b]], buf.at[b], sem
                ).wait()

        def run_scatters(g, buf):
            cps = [
                pltpu.async_copy(
                    buf.at[b], acc.at[idx_d.at[g * KF + b]], semS, add=True
                )
                for b in range(KF)
            ]
            for cp in cps:
                cp.wait()

        def step(g, buf, sem, obuf, osem):
            @pl.when(g + 1 < ngrp)
            def _():
                fire_gathers(g + 1, obuf, osem)

            drain_gathers(g, buf, sem)
            run_scatters(g, buf)

        fire_gathers(0, rows.at[0], semA)

        @pl.loop(0, ngrp)
        def _grp(g):
            @pl.when(g % 2 == 0)
            def _():
                step(g, rows.at[0], semA, rows.at[1], semB)

            @pl.when(g % 2 == 1)
            def _():
                step(g, rows.at[1], semB, rows.at[0], semA)

        plsc.subcore_barrier()
        pltpu.sync_copy(acc.at[pl.ds(sid * T_ROWS, T_ROWS)], stage)
        pltpu.sync_copy(stage, out.at[cid, pl.ds(sid * T_ROWS, T_ROWS)])

    return pl.kernel(
        body,
        out_type=jax.ShapeDtypeStruct((NC, N, d), jnp.float32),
        mesh=_mesh(),
        compiler_params=_SC_PARAMS,
        scratch_types=[
            pltpu.VMEM((C0, CH), jnp.int32),
            pltpu.VMEM((C0, CH), jnp.int32),
            pltpu.VMEM((2, KF, CH, d), jnp.float32),
            pltpu.VMEM((T_ROWS, d), jnp.float32),
            pltpu.VMEM_SHARED((N, d), jnp.float32),
            pltpu.SemaphoreType.DMA,
            pltpu.SemaphoreType.DMA,
            pltpu.SemaphoreType.DMA,
        ],
    )


_sc_degree = _make_sc_degree()
_sc_scatter_h = _make_sc_scatter(HP)
_sc_scatter_z = _make_sc_scatter(ZP)


def _tc_matmul(x, W1p):
    def body(x_ref, w_ref, o_ref):
        o_ref[...] = jnp.dot(x_ref[...], w_ref[...], preferred_element_type=jnp.float32)

    return pl.pallas_call(
        body,
        grid=(_GRID,),
        in_specs=[
            pl.BlockSpec((_ROW_BLK, D_FEAT), lambda i: (i, 0)),
            pl.BlockSpec((D_FEAT, HP), lambda i: (0, 0)),
        ],
        out_specs=pl.BlockSpec((_ROW_BLK, HP), lambda i: (i, 0)),
        out_shape=jax.ShapeDtypeStruct((N, HP), jnp.float32),
    )(x, W1p)


def _tc_scale(deg2, u1):
    def body(dref, u, dinv_ref, v_ref):
        dd = dref[...]
        deg = 1.0 + dd[0][:, :1] + dd[1][:, :1]
        dinv = 1.0 / jnp.sqrt(deg)
        dinv_ref[...] = dinv
        v_ref[...] = u[...] * dinv

    return pl.pallas_call(
        body,
        grid=(_GRID,),
        in_specs=[
            pl.BlockSpec((NC, _ROW_BLK, DW), lambda i: (0, i, 0)),
            pl.BlockSpec((_ROW_BLK, HP), lambda i: (i, 0)),
        ],
        out_specs=[
            pl.BlockSpec((_ROW_BLK, 1), lambda i: (i, 0)),
            pl.BlockSpec((_ROW_BLK, HP), lambda i: (i, 0)),
        ],
        out_shape=[
            jax.ShapeDtypeStruct((N, 1), jnp.float32),
            jax.ShapeDtypeStruct((N, HP), jnp.float32),
        ],
    )(deg2, u1)


def _tc_mid(dinv, s1, v1, W2p, b1r):
    def body(di, s_ref, v_ref, w_ref, bias_ref, o_ref):
        dinv_blk = di[...]
        ss = s_ref[...]
        h = dinv_blk * (ss[0] + ss[1] + v_ref[...]) + bias_ref[...]
        u2 = jnp.dot(h, w_ref[...], preferred_element_type=jnp.float32)
        o_ref[...] = u2 * dinv_blk

    return pl.pallas_call(
        body,
        grid=(_GRID,),
        in_specs=[
            pl.BlockSpec((_ROW_BLK, 1), lambda i: (i, 0)),
            pl.BlockSpec((NC, _ROW_BLK, HP), lambda i: (0, i, 0)),
            pl.BlockSpec((_ROW_BLK, HP), lambda i: (i, 0)),
            pl.BlockSpec((HP, ZP), lambda i: (0, 0)),
            pl.BlockSpec((1, HP), lambda i: (0, 0)),
        ],
        out_specs=pl.BlockSpec((_ROW_BLK, ZP), lambda i: (i, 0)),
        out_shape=jax.ShapeDtypeStruct((N, ZP), jnp.float32),
    )(dinv, s1, v1, W2p, b1r)


def _tc_final(dinv, s2, v2, b2r):
    def body(di, s_ref, v_ref, bias_ref, o_ref):
        ss = s_ref[...]
        zfull = di[...] * (ss[0] + ss[1] + v_ref[...]) + bias_ref[...]
        o_ref[...] = zfull[:, :Z]

    return pl.pallas_call(
        body,
        grid=(_GRID,),
        in_specs=[
            pl.BlockSpec((_ROW_BLK, 1), lambda i: (i, 0)),
            pl.BlockSpec((NC, _ROW_BLK, ZP), lambda i: (0, i, 0)),
            pl.BlockSpec((_ROW_BLK, ZP), lambda i: (i, 0)),
            pl.BlockSpec((1, ZP), lambda i: (0, 0)),
        ],
        out_specs=pl.BlockSpec((_ROW_BLK, Z), lambda i: (i, 0)),
        out_shape=jax.ShapeDtypeStruct((N, Z), jnp.float32),
    )(dinv, s2, v2, b2r)


def kernel(x, edge_index, W1, b1, W2, b2):
    e2 = edge_index.astype(jnp.int32)
    e3 = e2.reshape(2, NCHUNK, CH)
    W1p = jnp.pad(W1, ((0, 0), (0, HP - H)))
    W2p = jnp.pad(W2, ((0, HP - H), (0, ZP - Z)))
    b1r = jnp.pad(b1, (0, HP - H)).reshape(1, HP)
    b2r = jnp.pad(b2, (0, ZP - Z)).reshape(1, ZP)
    ones_c = jnp.ones((_DCH, DW), jnp.float32)
    zdw = jnp.zeros((T_ROWS, DW), jnp.float32)
    zh = jnp.zeros((T_ROWS, HP), jnp.float32)
    zz = jnp.zeros((T_ROWS, ZP), jnp.float32)

    deg2 = _sc_degree(e2, ones_c, zdw)
    u1 = _tc_matmul(x, W1p)
    dinv, v1 = _tc_scale(deg2, u1)
    s1 = _sc_scatter_h(e3, v1, zh)
    v2 = _tc_mid(dinv, s1, v1, W2p, b1r)
    s2 = _sc_scatter_z(e3, v2, zz)
    return _tc_final(dinv, s2, v2, b2r)
